# baseline (speedup 1.0000x reference)
import functools

import jax
import jax.numpy as jnp
from jax import lax
from jax.experimental import pallas as pl
from jax.experimental.pallas import tpu as pltpu
from jax.experimental.pallas import tpu_sc as plsc

N = 500
E = 2000
F = 128
NP = 512
WC = 128

NC = 2
NS = 16
NW = NS
EPW = 128


def _gather_body(h_hbm, cpad_hbm, row_hbm, col_hbm,
                 hrow_hbm, hcol_hbm, crow_hbm, ccol_hbm,
                 idx_r, idx_c, hbuf_r, hbuf_c, cbuf_r, cbuf_c, cpk_r, cpk_c,
                 sem_hr, sem_hc, sem_cr, sem_cc):
    wid = lax.axis_index("s")
    base = jnp.minimum(wid * EPW, E - EPW)
    ld_r = pltpu.async_copy(row_hbm.at[pl.ds(base, EPW)], idx_r, sem_hr)
    ld_c = pltpu.async_copy(col_hbm.at[pl.ds(base, EPW)], idx_c, sem_hc)
    ld_r.wait()
    cp_hr = pltpu.async_copy(h_hbm.at[idx_r], hbuf_r, sem_hr)
    cp_cr = pltpu.async_copy(cpad_hbm.at[idx_r], cbuf_r, sem_cr)
    ld_c.wait()
    cp_hc = pltpu.async_copy(h_hbm.at[idx_c], hbuf_c, sem_hc)
    cp_cc = pltpu.async_copy(cpad_hbm.at[idx_c], cbuf_c, sem_cc)
    cp_hr.wait()
    wb_hr = pltpu.async_copy(hbuf_r, hrow_hbm.at[pl.ds(base, EPW)], sem_hr)
    cp_hc.wait()
    wb_hc = pltpu.async_copy(hbuf_c, hcol_hbm.at[pl.ds(base, EPW)], sem_hc)
    cp_cr.wait()

    def _pack_r(r, carry):
        cpk_r[r, pl.ds(0, 16)] = cbuf_r[r, pl.ds(0, 16)]
        return carry

    lax.fori_loop(0, EPW, _pack_r, 0)
    wb_cr = pltpu.async_copy(cpk_r, crow_hbm.at[pl.ds(base, EPW)], sem_cr)
    cp_cc.wait()

    def _pack_c(r, carry):
        cpk_c[r, pl.ds(0, 16)] = cbuf_c[r, pl.ds(0, 16)]
        return carry

    lax.fori_loop(0, EPW, _pack_c, 0)
    wb_cc = pltpu.async_copy(cpk_c, ccol_hbm.at[pl.ds(base, EPW)], sem_cc)
    wb_hr.wait()
    wb_hc.wait()
    wb_cr.wait()
    wb_cc.wait()


@functools.cache
def _gather_call():
    mesh = plsc.VectorSubcoreMesh(core_axis_name="c", subcore_axis_name="s", num_cores=1)
    return pl.kernel(
        _gather_body,
        mesh=mesh,
        out_type=(jax.ShapeDtypeStruct((E, F), jnp.float32),
                  jax.ShapeDtypeStruct((E, F), jnp.float32),
                  jax.ShapeDtypeStruct((E, 16), jnp.float32),
                  jax.ShapeDtypeStruct((E, 16), jnp.float32)),
        scratch_types=[
            pltpu.VMEM((EPW,), jnp.int32),
            pltpu.VMEM((EPW,), jnp.int32),
            pltpu.VMEM((EPW, F), jnp.float32),
            pltpu.VMEM((EPW, F), jnp.float32),
            pltpu.VMEM((EPW, WC), jnp.float32),
            pltpu.VMEM((EPW, WC), jnp.float32),
            pltpu.VMEM((EPW, 16), jnp.float32),
            pltpu.VMEM((EPW, 16), jnp.float32),
            pltpu.SemaphoreType.DMA,
            pltpu.SemaphoreType.DMA,
            pltpu.SemaphoreType.DMA,
            pltpu.SemaphoreType.DMA,
        ],
    )


def _dense_body(hrow_ref, hcol_ref, crow_ref, ccol_ref, row2d_ref,
                h_ref, coord_ref, vel_ref,
                w1h_ref, w1c_ref, w1r_ref, b1_ref, w2_ref, b2_ref,
                wc1_ref, bc1_ref, wc2r_ref, bc2_ref,
                wn1h_ref, wn1a_ref, bn1_ref, wn2_ref, bn2_ref,
                hout_ref, cv_ref):
    f32 = jnp.float32
    bf16 = jnp.bfloat16
    hr = hrow_ref[...].astype(bf16)
    hc = hcol_ref[...].astype(bf16)
    cd = crow_ref[:, :3] - ccol_ref[:, :3]
    radial = jnp.sum(cd * cd, axis=1, keepdims=True)

    x = (jnp.dot(hr, w1h_ref[...].astype(bf16), preferred_element_type=f32)
         + jnp.dot(hc, w1c_ref[...].astype(bf16), preferred_element_type=f32)
         + radial * w1r_ref[...]
         + b1_ref[...])
    x = jnp.maximum(x, 0.0).astype(bf16)
    ef = jnp.maximum(
        jnp.dot(x, w2_ref[...].astype(bf16), preferred_element_type=f32)
        + b2_ref[...], 0.0)
    efb = ef.astype(bf16)

    c1 = jnp.maximum(
        jnp.dot(efb, wc1_ref[...].astype(bf16), preferred_element_type=f32)
        + bc1_ref[...], 0.0)
    cm = jnp.sum(c1 * wc2r_ref[...], axis=1, keepdims=True) + bc2_ref[0, 0]
    trans = jnp.clip(cd * cm, -100.0, 100.0)

    ones = jnp.ones((E, 1), bf16)
    payload = jnp.concatenate([efb, trans.astype(bf16), ones], axis=1)

    rowv = row2d_ref[...]
    niota = lax.broadcasted_iota(jnp.int32, (NP, E), 0)
    oh = jnp.where(niota == rowv, 1.0, 0.0).astype(bf16)
    agg = jnp.dot(oh, payload, preferred_element_type=f32)

    aggn = agg[:N, :F]
    ts = agg[:N, F:F + 3]
    cnt = agg[:N, F + 3:F + 4]
    aggc = ts / jnp.maximum(cnt, 1.0)

    v = vel_ref[...] + aggc * 0.125
    cv_ref[...] = jnp.concatenate(
        [coord_ref[...] + v * 0.125, v, jnp.zeros((N, 2), f32)], axis=1)

    hn = h_ref[...]
    y = jnp.maximum(
        jnp.dot(hn.astype(bf16), wn1h_ref[...].astype(bf16),
                preferred_element_type=f32)
        + jnp.dot(aggn.astype(bf16), wn1a_ref[...].astype(bf16),
                  preferred_element_type=f32)
        + bn1_ref[...], 0.0)
    hout_ref[...] = (hn
                     + jnp.dot(y.astype(bf16), wn2_ref[...].astype(bf16),
                               preferred_element_type=f32)
                     + bn2_ref[...])


_dense_call = pl.pallas_call(
    _dense_body,
    out_shape=(jax.ShapeDtypeStruct((N, F), jnp.float32),
               jax.ShapeDtypeStruct((N, 8), jnp.float32)),
)


def kernel(h, edge_index, coord, vel, vel_init,
           We1, be1, We2, be2, Wn1, bn1, Wn2, bn2, Wc1, bc1, Wc2, bc2):
    del vel_init
    f32 = jnp.float32
    row = edge_index[0].astype(jnp.int32)
    col = edge_index[1].astype(jnp.int32)
    cpad = jnp.zeros((N, WC), f32).at[:, :3].set(coord)

    hrow, hcol, crow, ccol = _gather_call()(h, cpad, row, col)

    h_new, cv = _dense_call(
        hrow, hcol, crow, ccol, row[None],
        h, coord, vel,
        We1[:F], We1[F:2 * F], We1[2 * F:2 * F + 1], be1[None],
        We2, be2[None], Wc1, bc1[None], Wc2.T, bc2[None],
        Wn1[:F], Wn1[F:], bn1[None], Wn2, bn2[None])

    return (h_new, cv[:, :3], cv[:, 3:6])


# speedup vs baseline: 1.1029x; 1.1029x over previous
import functools

import jax
import jax.numpy as jnp
from jax import lax
from jax.experimental import pallas as pl
from jax.experimental.pallas import tpu as pltpu
from jax.experimental.pallas import tpu_sc as plsc

N = 500
E = 2000
F = 128
NP = 512

NS = 16
EPW = 128


def _gather_body(h_hbm, row_hbm, col_hbm, hrow_hbm, hcol_hbm,
                 idx_r, idx_c, hbuf_r, hbuf_c, sem_r, sem_c):
    wid = lax.axis_index("s")
    base = jnp.minimum(wid * EPW, E - EPW)
    ld_r = pltpu.async_copy(row_hbm.at[pl.ds(base, EPW)], idx_r, sem_r)
    ld_c = pltpu.async_copy(col_hbm.at[pl.ds(base, EPW)], idx_c, sem_c)
    ld_r.wait()
    cp_r = pltpu.async_copy(h_hbm.at[idx_r], hbuf_r, sem_r)
    ld_c.wait()
    cp_c = pltpu.async_copy(h_hbm.at[idx_c], hbuf_c, sem_c)
    cp_r.wait()
    wb_r = pltpu.async_copy(hbuf_r, hrow_hbm.at[pl.ds(base, EPW)], sem_r)
    cp_c.wait()
    wb_c = pltpu.async_copy(hbuf_c, hcol_hbm.at[pl.ds(base, EPW)], sem_c)
    wb_r.wait()
    wb_c.wait()


@functools.cache
def _gather_call():
    mesh = plsc.VectorSubcoreMesh(core_axis_name="c", subcore_axis_name="s",
                                  num_cores=1)
    return pl.kernel(
        _gather_body,
        mesh=mesh,
        out_type=(jax.ShapeDtypeStruct((E, F), jnp.float32),
                  jax.ShapeDtypeStruct((E, F), jnp.float32)),
        scratch_types=[
            pltpu.VMEM((EPW,), jnp.int32),
            pltpu.VMEM((EPW,), jnp.int32),
            pltpu.VMEM((EPW, F), jnp.float32),
            pltpu.VMEM((EPW, F), jnp.float32),
            pltpu.SemaphoreType.DMA,
            pltpu.SemaphoreType.DMA,
        ],
    )


def _dense_body(hrow_ref, hcol_ref, row2d_ref, col2d_ref,
                h_ref, coord_ref, vel_ref,
                w1h_ref, w1c_ref, w1r_ref, b1_ref, w2_ref, b2_ref,
                wc1_ref, bc1_ref, wc2r_ref, bc2_ref,
                wn1h_ref, wn1a_ref, bn1_ref, wn2_ref, bn2_ref,
                hout_ref, cout_ref, vout_ref):
    f32 = jnp.float32
    bf16 = jnp.bfloat16
    hr = hrow_ref[...].astype(bf16)
    hc = hcol_ref[...].astype(bf16)

    rowv = row2d_ref[...]
    colv = col2d_ref[...]
    eiota = lax.broadcasted_iota(jnp.int32, (E, NP), 1)
    inc = (jnp.where(eiota == rowv.reshape(E, 1), 1.0, 0.0)
           - jnp.where(eiota == colv.reshape(E, 1), 1.0, 0.0))
    coordp = jnp.concatenate(
        [coord_ref[...], jnp.zeros((N, 13), f32)], axis=1)
    coordp = jnp.concatenate(
        [coordp, jnp.zeros((NP - N, 16), f32)], axis=0)
    cd16 = jnp.dot(inc, coordp, preferred_element_type=f32)
    cd = cd16[:, :3]
    radial = jnp.sum(cd * cd, axis=1, keepdims=True)

    x = (jnp.dot(hr, w1h_ref[...].astype(bf16), preferred_element_type=f32)
         + jnp.dot(hc, w1c_ref[...].astype(bf16), preferred_element_type=f32)
         + radial * w1r_ref[...]
         + b1_ref[...])
    x = jnp.maximum(x, 0.0).astype(bf16)
    ef = jnp.maximum(
        jnp.dot(x, w2_ref[...].astype(bf16), preferred_element_type=f32)
        + b2_ref[...], 0.0)
    efb = ef.astype(bf16)

    c1 = jnp.maximum(
        jnp.dot(efb, wc1_ref[...].astype(bf16), preferred_element_type=f32)
        + bc1_ref[...], 0.0)
    cm = jnp.sum(c1 * wc2r_ref[...], axis=1, keepdims=True) + bc2_ref[0, 0]
    trans = jnp.clip(cd * cm, -100.0, 100.0)

    ones = jnp.ones((E, 1), bf16)
    payload = jnp.concatenate([efb, trans.astype(bf16), ones], axis=1)

    niota = lax.broadcasted_iota(jnp.int32, (NP, E), 0)
    oh = jnp.where(niota == rowv, 1.0, 0.0).astype(bf16)
    agg = jnp.dot(oh, payload, preferred_element_type=f32)

    aggn = agg[:N, :F]
    ts = agg[:N, F:F + 3]
    cnt = agg[:N, F + 3:F + 4]
    aggc = ts / jnp.maximum(cnt, 1.0)

    v = vel_ref[...] + aggc * 0.125
    cout_ref[...] = coord_ref[...] + v * 0.125
    vout_ref[...] = v

    hn = h_ref[...]
    y = jnp.maximum(
        jnp.dot(hn.astype(bf16), wn1h_ref[...].astype(bf16),
                preferred_element_type=f32)
        + jnp.dot(aggn.astype(bf16), wn1a_ref[...].astype(bf16),
                  preferred_element_type=f32)
        + bn1_ref[...], 0.0)
    hout_ref[...] = (hn
                     + jnp.dot(y.astype(bf16), wn2_ref[...].astype(bf16),
                               preferred_element_type=f32)
                     + bn2_ref[...])


_dense_call = pl.pallas_call(
    _dense_body,
    out_shape=(jax.ShapeDtypeStruct((N, F), jnp.float32),
               jax.ShapeDtypeStruct((N, 3), jnp.float32),
               jax.ShapeDtypeStruct((N, 3), jnp.float32)),
)


def kernel(h, edge_index, coord, vel, vel_init,
           We1, be1, We2, be2, Wn1, bn1, Wn2, bn2, Wc1, bc1, Wc2, bc2):
    del vel_init
    row = edge_index[0].astype(jnp.int32)
    col = edge_index[1].astype(jnp.int32)

    hrow, hcol = _gather_call()(h, row, col)

    h_new, coord_new, v = _dense_call(
        hrow, hcol, row[None], col[None],
        h, coord, vel,
        We1[:F], We1[F:2 * F], We1[2 * F:2 * F + 1], be1[None],
        We2, be2[None], Wc1, bc1[None], Wc2.T, bc2[None],
        Wn1[:F], Wn1[F:], bn1[None], Wn2, bn2[None])

    return (h_new, coord_new, v)
